# baseline (speedup 1.0000x reference)
import functools

import jax
import jax.numpy as jnp
from jax import lax
from jax.experimental import pallas as pl
from jax.experimental.pallas import tpu as pltpu
from jax.experimental.pallas import tpu_sc as plsc

B = 16384
GMF_DIM = 16
MLP_DIM = 64

NC = 2
NS = 16
NW = NC * NS
BPW = B // NW
CHUNK = 128
NCHUNK = BPW // CHUNK


def _sc_gather(uid, iid, gu_t, gi_t, mu_t, mi_t):
    mesh = plsc.VectorSubcoreMesh(core_axis_name="c", subcore_axis_name="s")

    @functools.partial(
        pl.kernel,
        mesh=mesh,
        compiler_params=pltpu.CompilerParams(use_tc_tiling_on_sc=False),
        out_type=[
            jax.ShapeDtypeStruct((B, GMF_DIM), jnp.float32),
            jax.ShapeDtypeStruct((B, GMF_DIM), jnp.float32),
            jax.ShapeDtypeStruct((B, MLP_DIM), jnp.float32),
            jax.ShapeDtypeStruct((B, MLP_DIM), jnp.float32),
        ],
        scratch_types=[
            pltpu.VMEM((NCHUNK, CHUNK), jnp.int32),
            pltpu.VMEM((NCHUNK, CHUNK), jnp.int32),
            pltpu.VMEM((BPW, GMF_DIM), jnp.float32),
            pltpu.VMEM((BPW, GMF_DIM), jnp.float32),
            pltpu.VMEM((BPW, MLP_DIM), jnp.float32),
            pltpu.VMEM((BPW, MLP_DIM), jnp.float32),
            pltpu.SemaphoreType.DMA,
        ],
    )
    def body(uid_hbm, iid_hbm, gu_hbm, gi_hbm, mu_hbm, mi_hbm,
             out_gu, out_gi, out_mu, out_mi,
             uidx, iidx, gu_v, gi_v, mu_v, mi_v, sem):
        wid = lax.axis_index("s") * NC + lax.axis_index("c")
        base = wid * BPW
        for j in range(NCHUNK):
            pltpu.sync_copy(uid_hbm.at[pl.ds(base + j * CHUNK, CHUNK)], uidx.at[j])
            pltpu.sync_copy(iid_hbm.at[pl.ds(base + j * CHUNK, CHUNK)], iidx.at[j])
        copies = []
        for j in range(NCHUNK):
            sl = pl.ds(j * CHUNK, CHUNK)
            copies.append(pltpu.async_copy(gu_hbm.at[uidx.at[j]], gu_v.at[sl], sem))
            copies.append(pltpu.async_copy(gi_hbm.at[iidx.at[j]], gi_v.at[sl], sem))
            copies.append(pltpu.async_copy(mu_hbm.at[uidx.at[j]], mu_v.at[sl], sem))
            copies.append(pltpu.async_copy(mi_hbm.at[iidx.at[j]], mi_v.at[sl], sem))
        for c in copies:
            c.wait()
        pltpu.sync_copy(gu_v, out_gu.at[pl.ds(base, BPW)])
        pltpu.sync_copy(gi_v, out_gi.at[pl.ds(base, BPW)])
        pltpu.sync_copy(mu_v, out_mu.at[pl.ds(base, BPW)])
        pltpu.sync_copy(mi_v, out_mi.at[pl.ds(base, BPW)])

    return body(uid, iid, gu_t, gi_t, mu_t, mi_t)


def _tc_body(gu_ref, gi_ref, mu_ref, mi_ref, W0_ref, b0_ref, W1_ref, b1_ref,
             W2_ref, b2_ref, Wout_ref, bout_ref, out_ref):
    x = jnp.dot(mu_ref[...], W0_ref[:MLP_DIM, :], preferred_element_type=jnp.float32)
    x = x + jnp.dot(mi_ref[...], W0_ref[MLP_DIM:, :], preferred_element_type=jnp.float32)
    x = jnp.maximum(x + b0_ref[...], 0.0)
    x = jnp.maximum(jnp.dot(x, W1_ref[...], preferred_element_type=jnp.float32) + b1_ref[...], 0.0)
    x = jnp.maximum(jnp.dot(x, W2_ref[...], preferred_element_type=jnp.float32) + b2_ref[...], 0.0)
    g = gu_ref[...] * gi_ref[...]
    z = (jnp.dot(g, Wout_ref[:GMF_DIM, :], preferred_element_type=jnp.float32)
         + jnp.dot(x, Wout_ref[GMF_DIM:, :], preferred_element_type=jnp.float32))
    out_ref[...] = jax.nn.sigmoid(z + bout_ref[...])


def _tc_compute(gu, gi, mu, mi, W0, b0, W1, b1, W2, b2, Wout, bout):
    BLK = 2048
    grid = (B // BLK,)
    full = lambda shape: pl.BlockSpec(shape, lambda i: (0, 0))
    return pl.pallas_call(
        _tc_body,
        grid=grid,
        in_specs=[
            pl.BlockSpec((BLK, GMF_DIM), lambda i: (i, 0)),
            pl.BlockSpec((BLK, GMF_DIM), lambda i: (i, 0)),
            pl.BlockSpec((BLK, MLP_DIM), lambda i: (i, 0)),
            pl.BlockSpec((BLK, MLP_DIM), lambda i: (i, 0)),
            full((2 * MLP_DIM, 64)),
            full((1, 64)),
            full((64, 32)),
            full((1, 32)),
            full((32, GMF_DIM)),
            full((1, GMF_DIM)),
            full((32, 1)),
            full((1, 1)),
        ],
        out_specs=pl.BlockSpec((BLK, 1), lambda i: (i, 0)),
        out_shape=jax.ShapeDtypeStruct((B, 1), jnp.float32),
    )(gu, gi, mu, mi, W0, b0, W1, b1, W2, b2, Wout, bout)


def kernel(user_id, item_id, gmf_user_table, gmf_item_table, mlp_user_table,
           mlp_item_table, W0, b0, W1, b1, W2, b2, Wout, bout):
    uid = user_id.astype(jnp.int32)
    iid = item_id.astype(jnp.int32)
    gu, gi, mu, mi = _sc_gather(uid, iid, gmf_user_table, gmf_item_table,
                                mlp_user_table, mlp_item_table)
    return _tc_compute(gu, gi, mu, mi, W0, b0.reshape(1, -1), W1,
                       b1.reshape(1, -1), W2, b2.reshape(1, -1), Wout,
                       bout.reshape(1, -1))


# speedup vs baseline: 1.4408x; 1.4408x over previous
import functools

import jax
import jax.numpy as jnp
from jax import lax
from jax.experimental import pallas as pl
from jax.experimental.pallas import tpu as pltpu
from jax.experimental.pallas import tpu_sc as plsc

B = 16384
GMF_DIM = 16
MLP_DIM = 64

NC = 2
NS = 16
NW = NC * NS
BPW = B // NW
L = 16
CH = 128
NCH = BPW // CH


def _sc_gather(uid, iid, gu_t, gi_t, mu_t, mi_t):
    mesh = plsc.VectorSubcoreMesh(core_axis_name="c", subcore_axis_name="s")

    @functools.partial(
        pl.kernel,
        mesh=mesh,
        out_type=[
            jax.ShapeDtypeStruct((B, 128), jnp.float32),
            jax.ShapeDtypeStruct((B, 128), jnp.float32),
        ],
        scratch_types=[
            pltpu.VMEM((BPW,), jnp.int32),
            pltpu.VMEM((BPW,), jnp.int32),
            pltpu.VMEM((2 * CH, GMF_DIM), jnp.float32),
            pltpu.VMEM((2 * CH, MLP_DIM), jnp.float32),
            pltpu.VMEM((CH, 128), jnp.float32),
            pltpu.SemaphoreType.DMA,
        ],
    )
    def body(uid_hbm, iid_hbm, gu_hbm, gi_hbm, mu_hbm, mi_hbm,
             out_mlp, out_gmf,
             uidx_v, iidx_v, gbuf, mbuf, stage, sem):
        wid = lax.axis_index("s") * NC + lax.axis_index("c")
        base = wid * BPW
        pltpu.sync_copy(uid_hbm.at[pl.ds(base, BPW)], uidx_v)
        pltpu.sync_copy(iid_hbm.at[pl.ds(base, BPW)], iidx_v)

        for ch in range(NCH):
            def gstep(g, carry, ch=ch):
                uvec = uidx_v[pl.ds(ch * CH + g * L, L)]
                ivec = iidx_v[pl.ds(ch * CH + g * L, L)]
                for l in range(L):
                    u = uvec[l]
                    i = ivec[l]
                    k = g * L + l
                    pltpu.async_copy(mu_hbm.at[pl.ds(u, 1)], mbuf.at[pl.ds(2 * k, 1)], sem)
                    pltpu.async_copy(mi_hbm.at[pl.ds(i, 1)], mbuf.at[pl.ds(2 * k + 1, 1)], sem)
                    pltpu.async_copy(gu_hbm.at[pl.ds(u, 1)], gbuf.at[pl.ds(2 * k, 1)], sem)
                    pltpu.async_copy(gi_hbm.at[pl.ds(i, 1)], gbuf.at[pl.ds(2 * k + 1, 1)], sem)
                return carry

            lax.fori_loop(0, CH // L, gstep, 0)
            pltpu.make_async_copy(mu_hbm.at[pl.ds(0, 2 * CH)], mbuf, sem).wait()
            pltpu.make_async_copy(gu_hbm.at[pl.ds(0, 2 * CH)], gbuf, sem).wait()

            def mstep(k, carry):
                for c in range(MLP_DIM // L):
                    stage[k, pl.ds(c * L, L)] = mbuf[2 * k, pl.ds(c * L, L)]
                    stage[k, pl.ds(64 + c * L, L)] = mbuf[2 * k + 1, pl.ds(c * L, L)]
                return carry

            lax.fori_loop(0, CH, mstep, 0)
            pltpu.sync_copy(stage, out_mlp.at[pl.ds(base + ch * CH, CH)])

            def pstep(k, carry):
                a = gbuf[2 * k, pl.ds(0, L)]
                b = gbuf[2 * k + 1, pl.ds(0, L)]
                stage[k, pl.ds(0, L)] = a * b
                return carry

            lax.fori_loop(0, CH, pstep, 0)
            pltpu.sync_copy(stage, out_gmf.at[pl.ds(base + ch * CH, CH)])

    return body(uid, iid, gu_t, gi_t, mu_t, mi_t)


def _tc_body(mlp_ref, gmf_ref, W0_ref, b0_ref, W1_ref, b1_ref,
             W2_ref, b2_ref, Wout_ref, bout_ref, out_ref):
    x = jnp.dot(mlp_ref[...], W0_ref[...], preferred_element_type=jnp.float32)
    x = jnp.maximum(x + b0_ref[...], 0.0)
    x = jnp.maximum(jnp.dot(x, W1_ref[...], preferred_element_type=jnp.float32) + b1_ref[...], 0.0)
    x = jnp.maximum(jnp.dot(x, W2_ref[...], preferred_element_type=jnp.float32) + b2_ref[...], 0.0)
    z = (jnp.dot(gmf_ref[:, :GMF_DIM], Wout_ref[:GMF_DIM, :], preferred_element_type=jnp.float32)
         + jnp.dot(x, Wout_ref[GMF_DIM:, :], preferred_element_type=jnp.float32))
    out_ref[...] = jax.nn.sigmoid(z + bout_ref[...])


def _tc_compute(mlp2d, gmf2d, W0, b0, W1, b1, W2, b2, Wout, bout):
    BLK = 2048
    grid = (B // BLK,)
    full = lambda shape: pl.BlockSpec(shape, lambda i: (0, 0))
    return pl.pallas_call(
        _tc_body,
        grid=grid,
        in_specs=[
            pl.BlockSpec((BLK, 128), lambda i: (i, 0)),
            pl.BlockSpec((BLK, 128), lambda i: (i, 0)),
            full((2 * MLP_DIM, 64)),
            full((1, 64)),
            full((64, 32)),
            full((1, 32)),
            full((32, GMF_DIM)),
            full((1, GMF_DIM)),
            full((32, 1)),
            full((1, 1)),
        ],
        out_specs=pl.BlockSpec((BLK, 1), lambda i: (i, 0)),
        out_shape=jax.ShapeDtypeStruct((B, 1), jnp.float32),
    )(mlp2d, gmf2d, W0, b0, W1, b1, W2, b2, Wout, bout)


def kernel(user_id, item_id, gmf_user_table, gmf_item_table, mlp_user_table,
           mlp_item_table, W0, b0, W1, b1, W2, b2, Wout, bout):
    uid = user_id.astype(jnp.int32)
    iid = item_id.astype(jnp.int32)
    out_mlp, out_gmf = _sc_gather(uid, iid, gmf_user_table, gmf_item_table,
                                  mlp_user_table, mlp_item_table)
    return _tc_compute(out_mlp, out_gmf, W0, b0.reshape(1, -1), W1,
                       b1.reshape(1, -1), W2, b2.reshape(1, -1), Wout,
                       bout.reshape(1, -1))
